# baseline (speedup 1.0000x reference)
import jax
import jax.numpy as jnp
from jax.experimental import pallas as pl


_S_BLK = 512


def _add_kernel(x_ref, pos_ref, out_ref):
    pos = pos_ref[...]
    d = pos.shape[-1]
    out_ref[:, :d] = x_ref[:, :d] + pos
    out_ref[:, d:] = x_ref[:, d:] + pos


def kernel(x, pos_emb):
    seq_len, batch, d_model = x.shape
    x2 = x.reshape(seq_len, batch * d_model)
    grid = (seq_len // _S_BLK,)
    out = pl.pallas_call(
        _add_kernel,
        grid=grid,
        in_specs=[
            pl.BlockSpec((_S_BLK, batch * d_model), lambda i: (i, 0)),
            pl.BlockSpec((_S_BLK, d_model), lambda i: (i, 0)),
        ],
        out_specs=pl.BlockSpec((_S_BLK, batch * d_model), lambda i: (i, 0)),
        out_shape=jax.ShapeDtypeStruct((seq_len, batch * d_model), x.dtype),
    )(x2, pos_emb[:seq_len])
    return out.reshape(seq_len, batch, d_model)


# speedup vs baseline: 3.7206x; 3.7206x over previous
import jax
import jax.numpy as jnp
from jax.experimental import pallas as pl


_S_BLK = 1024


def _add_kernel(x_ref, pos_ref, out_ref):
    pos = pos_ref[...]
    out_ref[...] = x_ref[...] + pos[:, None, :]


def kernel(x, pos_emb):
    seq_len, batch, d_model = x.shape
    grid = (seq_len // _S_BLK,)
    return pl.pallas_call(
        _add_kernel,
        grid=grid,
        in_specs=[
            pl.BlockSpec((_S_BLK, batch, d_model), lambda i: (i, 0, 0)),
            pl.BlockSpec((_S_BLK, d_model), lambda i: (i, 0)),
        ],
        out_specs=pl.BlockSpec((_S_BLK, batch, d_model), lambda i: (i, 0, 0)),
        out_shape=jax.ShapeDtypeStruct((seq_len, batch, d_model), x.dtype),
    )(x, pos_emb[:seq_len])
